# Initial kernel scaffold; baseline (speedup 1.0000x reference)
import jax
import jax.numpy as jnp
from jax.experimental import pallas as pl


def kernel(x, table):
    raise NotImplementedError("write your pallas kernel here")


# speedup vs baseline: 5.0331x; 5.0331x over previous
import functools

import jax
import jax.numpy as jnp
from jax import lax
from jax.experimental import pallas as pl
from jax.experimental.pallas import tpu as pltpu
from jax.experimental.pallas import tpu_sc as plsc

_NC = 2
_NS = 16
_NW = _NC * _NS
_CHUNK = 128


@functools.lru_cache(maxsize=None)
def _build(n, d):
    per_w = n // _NW
    nch = per_w // _CHUNK
    mesh = plsc.VectorSubcoreMesh(core_axis_name="c", subcore_axis_name="s")

    @functools.partial(
        pl.kernel,
        out_type=jax.ShapeDtypeStruct((n, d), jnp.float32),
        mesh=mesh,
        compiler_params=pltpu.CompilerParams(use_tc_tiling_on_sc=False),
        scratch_types=[
            pltpu.VMEM((per_w,), jnp.int32),
            pltpu.VMEM((_CHUNK, d), jnp.float32),
            pltpu.SemaphoreType.DMA,
        ],
    )
    def grab(idx_hbm, table_hbm, out_hbm, idx_v, rows, sem):
        wid = lax.axis_index("s") * _NC + lax.axis_index("c")
        base = wid * per_w
        pltpu.sync_copy(idx_hbm.at[pl.ds(base, per_w)], idx_v)

        def body(j, carry):
            start = j * _CHUNK
            pltpu.async_copy(
                table_hbm.at[idx_v.at[pl.ds(start, _CHUNK)]], rows, sem
            ).wait()
            pltpu.sync_copy(rows, out_hbm.at[pl.ds(base + start, _CHUNK)])
            return carry

        lax.fori_loop(0, nch, body, 0)

    return grab


@jax.jit
def kernel(x, table):
    b, h = x.shape
    d = table.shape[1]
    idx = x.reshape(-1).astype(jnp.int32)
    out = _build(b * h, d)(idx, table)
    return out.reshape(b, h, d)
